# baseline (speedup 1.0000x reference)
import functools

import jax
import jax.numpy as jnp
from jax import lax
from jax.experimental import pallas as pl
from jax.experimental.pallas import tpu as pltpu
from jax.experimental.pallas import tpu_sc as plsc

N = 10000
F = 128
E = 320000
NC, NS = 2, 16
NW = NC * NS
CH = 128
CPT = 160
EPAD = NS * CPT * CH
S0 = 40
S1 = CPT - S0
SMX = max(S0, S1)
NP = 10240
RT = NP // NS

_sc_mesh = plsc.VectorSubcoreMesh(
    core_axis_name="c", subcore_axis_name="s", num_cores=NC, num_subcores=NS)


@functools.partial(
    pl.kernel,
    out_type=jax.ShapeDtypeStruct((NC, NP, 16), jnp.float32),
    mesh=_sc_mesh,
    scratch_types=[
        pltpu.VMEM((SMX, CH), jnp.int32),
        pltpu.VMEM((CH, 16), jnp.float32),
        pltpu.VMEM_SHARED((NP, 16), jnp.float32),
    ],
)
def _degree_kernel(col_hbm, ones_hbm, z16_hbm, hist_hbm, idx_v, ones_v, acc):
    cid = lax.axis_index("c")
    sid = lax.axis_index("s")
    n = jnp.where(cid == 0, S0, S1)
    pltpu.sync_copy(z16_hbm.at[pl.ds(sid * RT, RT)], acc.at[pl.ds(sid * RT, RT)])
    pltpu.sync_copy(ones_hbm, ones_v)

    @pl.when(cid == 0)
    def _():
        pltpu.sync_copy(col_hbm.at[sid, pl.ds(0, S0)], idx_v.at[pl.ds(0, S0)])

    @pl.when(cid == 1)
    def _():
        pltpu.sync_copy(col_hbm.at[sid, pl.ds(S0, S1)], idx_v.at[pl.ds(0, S1)])

    plsc.subcore_barrier()

    def body(j, c):
        pltpu.sync_copy(ones_v, acc.at[idx_v.at[j]], add=True)
        return c

    lax.fori_loop(0, n, body, 0)
    plsc.subcore_barrier()
    pltpu.sync_copy(acc.at[pl.ds(sid * RT, RT)],
                    hist_hbm.at[cid, pl.ds(sid * RT, RT)])


@functools.partial(
    pl.kernel,
    out_type=jax.ShapeDtypeStruct((NC, NP, F), jnp.float32),
    mesh=_sc_mesh,
    scratch_types=[
        pltpu.VMEM((SMX, CH), jnp.int32),
        pltpu.VMEM((SMX, CH), jnp.int32),
        pltpu.VMEM((CH, F), jnp.float32),
        pltpu.VMEM_SHARED((NP, F), jnp.float32),
    ],
)
def _prop_kernel(xs_hbm, row_hbm, col_hbm, z128_hbm, g_hbm,
                 rowi_v, coli_v, rows_v, acc):
    cid = lax.axis_index("c")
    sid = lax.axis_index("s")
    n = jnp.where(cid == 0, S0, S1)
    pltpu.sync_copy(z128_hbm.at[pl.ds(sid * RT, RT)], acc.at[pl.ds(sid * RT, RT)])

    @pl.when(cid == 0)
    def _():
        pltpu.sync_copy(row_hbm.at[sid, pl.ds(0, S0)], rowi_v.at[pl.ds(0, S0)])
        pltpu.sync_copy(col_hbm.at[sid, pl.ds(0, S0)], coli_v.at[pl.ds(0, S0)])

    @pl.when(cid == 1)
    def _():
        pltpu.sync_copy(row_hbm.at[sid, pl.ds(S0, S1)], rowi_v.at[pl.ds(0, S1)])
        pltpu.sync_copy(col_hbm.at[sid, pl.ds(S0, S1)], coli_v.at[pl.ds(0, S1)])

    plsc.subcore_barrier()

    def body(j, c):
        pltpu.sync_copy(xs_hbm.at[rowi_v.at[j]], rows_v)
        pltpu.sync_copy(rows_v, acc.at[coli_v.at[j]], add=True)
        return c

    lax.fori_loop(0, n, body, 0)
    plsc.subcore_barrier()
    pltpu.sync_copy(acc.at[pl.ds(sid * RT, RT)],
                    g_hbm.at[cid, pl.ds(sid * RT, RT)])


_BLK = 2000


def _scale_body(hist_ref, x_ref, xs_ref, dinv_ref):
    deg = hist_ref[0, :, 0:1] + hist_ref[1, :, 0:1] + 1.0
    dinv = lax.rsqrt(deg)
    xs_ref[...] = dinv * x_ref[...]
    dinv_ref[...] = jnp.broadcast_to(dinv, (_BLK, 16))


def _mid_body(g_ref, xs_ref, dinv_ref, w_ref, b_ref, xs2_ref):
    dinv = dinv_ref[:, 0:1]
    p = dinv * (g_ref[0] + g_ref[1] + xs_ref[...])
    h = jnp.dot(p, w_ref[...], preferred_element_type=jnp.float32) + b_ref[...]
    xs2_ref[...] = dinv * h


def _out_body(g_ref, xs2_ref, dinv_ref, w2_ref, b2_ref, w3_ref, b3_ref,
              m_ref, s_ref):
    dinv = dinv_ref[:, 0:1]
    p = dinv * (g_ref[0] + g_ref[1] + xs2_ref[...])
    m_ref[...] = jnp.maximum(
        jnp.dot(p, w2_ref[...], preferred_element_type=jnp.float32) + b2_ref[...], 0.0)
    s_ref[...] = jnp.maximum(
        jnp.dot(p, w3_ref[...], preferred_element_type=jnp.float32) + b3_ref[...], 0.0)


def _row_spec(i):
    return (i, 0)


def _pair_spec(i):
    return (0, i, 0)


def _full_spec(i):
    return (0, 0)


def kernel(x, edge_index, W1, b1, W2, b2, W3, b3):
    ei = edge_index.astype(jnp.int32)
    pad = EPAD - E
    row3 = jnp.concatenate([ei[0], jnp.zeros((pad,), jnp.int32)]).reshape(NS, CPT, CH)
    col3 = jnp.concatenate([ei[1], jnp.full((pad,), NP - 1, jnp.int32)]).reshape(NS, CPT, CH)
    ones16 = jnp.concatenate(
        [jnp.ones((CH, 1), jnp.float32), jnp.zeros((CH, 15), jnp.float32)], axis=1)
    z16 = jnp.zeros((NP, 16), jnp.float32)
    z128 = jnp.zeros((NP, F), jnp.float32)
    b1r, b2r, b3r = b1.reshape(1, F), b2.reshape(1, F), b3.reshape(1, F)

    hist = _degree_kernel(col3, ones16, z16)

    grid = (N // _BLK,)
    xs1, dinv16 = pl.pallas_call(
        _scale_body,
        grid=grid,
        in_specs=[pl.BlockSpec((NC, _BLK, 16), _pair_spec),
                  pl.BlockSpec((_BLK, F), _row_spec)],
        out_specs=[pl.BlockSpec((_BLK, F), _row_spec),
                   pl.BlockSpec((_BLK, 16), _row_spec)],
        out_shape=[jax.ShapeDtypeStruct((N, F), jnp.float32),
                   jax.ShapeDtypeStruct((N, 16), jnp.float32)],
    )(hist, x)

    g1 = _prop_kernel(xs1, row3, col3, z128)

    xs2 = pl.pallas_call(
        _mid_body,
        grid=grid,
        in_specs=[pl.BlockSpec((NC, _BLK, F), _pair_spec),
                  pl.BlockSpec((_BLK, F), _row_spec),
                  pl.BlockSpec((_BLK, 16), _row_spec),
                  pl.BlockSpec((F, F), _full_spec),
                  pl.BlockSpec((1, F), _full_spec)],
        out_specs=pl.BlockSpec((_BLK, F), _row_spec),
        out_shape=jax.ShapeDtypeStruct((N, F), jnp.float32),
    )(g1, xs1, dinv16, W1, b1r)

    g2 = _prop_kernel(xs2, row3, col3, z128)

    m, s = pl.pallas_call(
        _out_body,
        grid=grid,
        in_specs=[pl.BlockSpec((NC, _BLK, F), _pair_spec),
                  pl.BlockSpec((_BLK, F), _row_spec),
                  pl.BlockSpec((_BLK, 16), _row_spec),
                  pl.BlockSpec((F, F), _full_spec),
                  pl.BlockSpec((1, F), _full_spec),
                  pl.BlockSpec((F, F), _full_spec),
                  pl.BlockSpec((1, F), _full_spec)],
        out_specs=[pl.BlockSpec((_BLK, F), _row_spec),
                   pl.BlockSpec((_BLK, F), _row_spec)],
        out_shape=[jax.ShapeDtypeStruct((N, F), jnp.float32),
                   jax.ShapeDtypeStruct((N, F), jnp.float32)],
    )(g2, xs2, dinv16, W2, b2r, W3, b3r)

    return (m, s)


# speedup vs baseline: 1.1303x; 1.1303x over previous
import functools

import jax
import jax.numpy as jnp
from jax import lax
from jax.experimental import pallas as pl
from jax.experimental.pallas import tpu as pltpu
from jax.experimental.pallas import tpu_sc as plsc

N = 10000
F = 128
E = 320000
NC, NS = 2, 16
NW = NC * NS
CH = 128
CPT = 160
EPAD = NS * CPT * CH
S0 = 104
S1 = CPT - S0
SMX = max(S0, S1)
NP = 10240
RT = NP // NS

_sc_mesh = plsc.VectorSubcoreMesh(
    core_axis_name="c", subcore_axis_name="s", num_cores=NC, num_subcores=NS)


@functools.partial(
    pl.kernel,
    out_type=jax.ShapeDtypeStruct((NC, NP, 16), jnp.float32),
    mesh=_sc_mesh,
    scratch_types=[
        pltpu.VMEM((SMX, CH), jnp.int32),
        pltpu.VMEM((CH, 16), jnp.float32),
        pltpu.VMEM_SHARED((NP, 16), jnp.float32),
    ],
)
def _degree_kernel(col0_hbm, col1_hbm, ones_hbm, z16_hbm, hist_hbm,
                   idx_v, ones_v, acc):
    cid = lax.axis_index("c")
    sid = lax.axis_index("s")
    pltpu.sync_copy(z16_hbm.at[pl.ds(sid * RT, RT)], acc.at[pl.ds(sid * RT, RT)])
    pltpu.sync_copy(ones_hbm, ones_v)
    plsc.subcore_barrier()

    def run(col_hbm, nch):
        pltpu.sync_copy(col_hbm.at[sid], idx_v.at[pl.ds(0, nch)])

        def body(j, c):
            pltpu.sync_copy(ones_v, acc.at[idx_v.at[j]], add=True)
            return c

        lax.fori_loop(0, nch, body, 0)

    @pl.when(cid == 0)
    def _():
        run(col0_hbm, S0)

    @pl.when(cid == 1)
    def _():
        run(col1_hbm, S1)

    plsc.subcore_barrier()
    pltpu.sync_copy(acc.at[pl.ds(sid * RT, RT)],
                    hist_hbm.at[cid, pl.ds(sid * RT, RT)])


@functools.partial(
    pl.kernel,
    out_type=jax.ShapeDtypeStruct((NC, NP, F), jnp.float32),
    mesh=_sc_mesh,
    scratch_types=[
        pltpu.VMEM((SMX, CH), jnp.int32),
        pltpu.VMEM((SMX, CH), jnp.int32),
        pltpu.VMEM((CH, F), jnp.float32),
        pltpu.VMEM_SHARED((NP, F), jnp.float32),
    ],
)
def _prop_kernel(xs_hbm, row0_hbm, col0_hbm, row1_hbm, col1_hbm, z128_hbm,
                 g_hbm, rowi_v, coli_v, rows_v, acc):
    cid = lax.axis_index("c")
    sid = lax.axis_index("s")
    pltpu.sync_copy(z128_hbm.at[pl.ds(sid * RT, RT)], acc.at[pl.ds(sid * RT, RT)])
    plsc.subcore_barrier()

    def run(row_hbm, col_hbm, nch):
        pltpu.sync_copy(row_hbm.at[sid], rowi_v.at[pl.ds(0, nch)])
        pltpu.sync_copy(col_hbm.at[sid], coli_v.at[pl.ds(0, nch)])

        def body(j, c):
            pltpu.sync_copy(xs_hbm.at[rowi_v.at[j]], rows_v)
            pltpu.sync_copy(rows_v, acc.at[coli_v.at[j]], add=True)
            return c

        lax.fori_loop(0, nch, body, 0)

    @pl.when(cid == 0)
    def _():
        run(row0_hbm, col0_hbm, S0)

    @pl.when(cid == 1)
    def _():
        run(row1_hbm, col1_hbm, S1)

    plsc.subcore_barrier()
    pltpu.sync_copy(acc.at[pl.ds(sid * RT, RT)],
                    g_hbm.at[cid, pl.ds(sid * RT, RT)])


_BLK = 2000


def _scale_body(hist_ref, x_ref, xs_ref, dinv_ref):
    deg = hist_ref[0, :, 0:1] + hist_ref[1, :, 0:1] + 1.0
    dinv = lax.rsqrt(deg)
    xs_ref[...] = dinv * x_ref[...]
    dinv_ref[...] = jnp.broadcast_to(dinv, (_BLK, 16))


def _mid_body(g_ref, xs_ref, dinv_ref, w_ref, b_ref, xs2_ref):
    dinv = dinv_ref[:, 0:1]
    p = dinv * (g_ref[0] + g_ref[1] + xs_ref[...])
    h = jnp.dot(p, w_ref[...], preferred_element_type=jnp.float32) + b_ref[...]
    xs2_ref[...] = dinv * h


def _out_body(g_ref, xs2_ref, dinv_ref, w2_ref, b2_ref, w3_ref, b3_ref,
              m_ref, s_ref):
    dinv = dinv_ref[:, 0:1]
    p = dinv * (g_ref[0] + g_ref[1] + xs2_ref[...])
    m_ref[...] = jnp.maximum(
        jnp.dot(p, w2_ref[...], preferred_element_type=jnp.float32) + b2_ref[...], 0.0)
    s_ref[...] = jnp.maximum(
        jnp.dot(p, w3_ref[...], preferred_element_type=jnp.float32) + b3_ref[...], 0.0)


def _row_spec(i):
    return (i, 0)


def _pair_spec(i):
    return (0, i, 0)


def _full_spec(i):
    return (0, 0)


def kernel(x, edge_index, W1, b1, W2, b2, W3, b3):
    ei = edge_index.astype(jnp.int32)
    pad = EPAD - E
    row3 = jnp.concatenate([ei[0], jnp.zeros((pad,), jnp.int32)]).reshape(NS, CPT, CH)
    col3 = jnp.concatenate([ei[1], jnp.full((pad,), NP - 1, jnp.int32)]).reshape(NS, CPT, CH)
    row0, row1 = row3[:, :S0], row3[:, S0:]
    col0, col1 = col3[:, :S0], col3[:, S0:]
    ones16 = jnp.concatenate(
        [jnp.ones((CH, 1), jnp.float32), jnp.zeros((CH, 15), jnp.float32)], axis=1)
    z16 = jnp.zeros((NP, 16), jnp.float32)
    z128 = jnp.zeros((NP, F), jnp.float32)
    b1r, b2r, b3r = b1.reshape(1, F), b2.reshape(1, F), b3.reshape(1, F)

    hist = _degree_kernel(col0, col1, ones16, z16)

    grid = (N // _BLK,)
    xs1, dinv16 = pl.pallas_call(
        _scale_body,
        grid=grid,
        in_specs=[pl.BlockSpec((NC, _BLK, 16), _pair_spec),
                  pl.BlockSpec((_BLK, F), _row_spec)],
        out_specs=[pl.BlockSpec((_BLK, F), _row_spec),
                   pl.BlockSpec((_BLK, 16), _row_spec)],
        out_shape=[jax.ShapeDtypeStruct((N, F), jnp.float32),
                   jax.ShapeDtypeStruct((N, 16), jnp.float32)],
    )(hist, x)

    g1 = _prop_kernel(xs1, row0, col0, row1, col1, z128)

    xs2 = pl.pallas_call(
        _mid_body,
        grid=grid,
        in_specs=[pl.BlockSpec((NC, _BLK, F), _pair_spec),
                  pl.BlockSpec((_BLK, F), _row_spec),
                  pl.BlockSpec((_BLK, 16), _row_spec),
                  pl.BlockSpec((F, F), _full_spec),
                  pl.BlockSpec((1, F), _full_spec)],
        out_specs=pl.BlockSpec((_BLK, F), _row_spec),
        out_shape=jax.ShapeDtypeStruct((N, F), jnp.float32),
    )(g1, xs1, dinv16, W1, b1r)

    g2 = _prop_kernel(xs2, row0, col0, row1, col1, z128)

    m, s = pl.pallas_call(
        _out_body,
        grid=grid,
        in_specs=[pl.BlockSpec((NC, _BLK, F), _pair_spec),
                  pl.BlockSpec((_BLK, F), _row_spec),
                  pl.BlockSpec((_BLK, 16), _row_spec),
                  pl.BlockSpec((F, F), _full_spec),
                  pl.BlockSpec((1, F), _full_spec),
                  pl.BlockSpec((F, F), _full_spec),
                  pl.BlockSpec((1, F), _full_spec)],
        out_specs=[pl.BlockSpec((_BLK, F), _row_spec),
                   pl.BlockSpec((_BLK, F), _row_spec)],
        out_shape=[jax.ShapeDtypeStruct((N, F), jnp.float32),
                   jax.ShapeDtypeStruct((N, F), jnp.float32)],
    )(g2, xs2, dinv16, W2, b2r, W3, b3r)

    return (m, s)


# speedup vs baseline: 3.7222x; 3.2932x over previous
import functools

import jax
import jax.numpy as jnp
from jax import lax
from jax.experimental import pallas as pl
from jax.experimental.pallas import tpu as pltpu
from jax.experimental.pallas import tpu_sc as plsc

N = 10000
F = 128
E = 320000
NC, NS = 2, 16
NW = NC * NS
CH = 128
CPW = 80
EPAD = NW * CPW * CH
HALF = CPW // 2
NP = 10240
NJ = NP - N
RT = NP // NS

_sc_mesh = plsc.VectorSubcoreMesh(
    core_axis_name="c", subcore_axis_name="s", num_cores=NC, num_subcores=NS)


@functools.partial(
    pl.kernel,
    out_type=jax.ShapeDtypeStruct((NC, NP, 16), jnp.float32),
    mesh=_sc_mesh,
    scratch_types=[
        pltpu.VMEM((CPW, CH), jnp.int32),
        pltpu.VMEM((CH, 16), jnp.float32),
        pltpu.VMEM_SHARED((NP, 16), jnp.float32),
    ],
)
def _degree_kernel(col_hbm, ones_hbm, z16_hbm, hist_hbm, idx_v, ones_v, acc):
    cid = lax.axis_index("c")
    sid = lax.axis_index("s")
    wid = sid * NC + cid
    pltpu.sync_copy(z16_hbm.at[pl.ds(sid * RT, RT)], acc.at[pl.ds(sid * RT, RT)])
    pltpu.sync_copy(ones_hbm, ones_v)
    pltpu.sync_copy(col_hbm.at[wid], idx_v)
    plsc.subcore_barrier()

    def body(j, c):
        pltpu.sync_copy(ones_v, acc.at[idx_v.at[j]], add=True)
        return c

    lax.fori_loop(0, CPW, body, 0)
    plsc.subcore_barrier()
    pltpu.sync_copy(acc.at[pl.ds(sid * RT, RT)],
                    hist_hbm.at[cid, pl.ds(sid * RT, RT)])


@functools.partial(
    pl.kernel,
    out_type=jax.ShapeDtypeStruct((NC, NP, F), jnp.float32),
    mesh=_sc_mesh,
    scratch_types=[
        pltpu.VMEM((HALF, CH), jnp.int32),
        pltpu.VMEM((HALF, CH), jnp.int32),
        pltpu.VMEM((CH, F), jnp.float32),
        pltpu.VMEM((CH, F), jnp.float32),
        pltpu.SemaphoreType.DMA,
        pltpu.SemaphoreType.DMA,
        pltpu.VMEM_SHARED((NP, F), jnp.float32),
    ],
)
def _prop_kernel(xs_hbm, row_hbm, col_hbm, z128_hbm, g_hbm,
                 rowi_v, coli_v, rows0_v, rows1_v, sem0, sem1, acc):
    cid = lax.axis_index("c")
    sid = lax.axis_index("s")
    wid = sid * NC + cid
    pltpu.sync_copy(z128_hbm.at[pl.ds(sid * RT, RT)], acc.at[pl.ds(sid * RT, RT)])
    plsc.subcore_barrier()

    def half_body(h, carry):
        pltpu.sync_copy(row_hbm.at[wid, pl.ds(h * HALF, HALF)], rowi_v)
        pltpu.sync_copy(col_hbm.at[wid, pl.ds(h * HALF, HALF)], coli_v)
        pltpu.async_copy(xs_hbm.at[rowi_v.at[0]], rows0_v, sem0)

        def body(i, c):
            j = 2 * i
            pltpu.async_copy(xs_hbm.at[rowi_v.at[j + 1]], rows1_v, sem1)
            pltpu.make_async_copy(xs_hbm.at[rowi_v.at[j]], rows0_v, sem0).wait()
            pltpu.sync_copy(rows0_v, acc.at[coli_v.at[j]], add=True)
            pltpu.async_copy(xs_hbm.at[rowi_v.at[j + 2]], rows0_v, sem0)
            pltpu.make_async_copy(xs_hbm.at[rowi_v.at[j + 1]], rows1_v, sem1).wait()
            pltpu.sync_copy(rows1_v, acc.at[coli_v.at[j + 1]], add=True)
            return c

        lax.fori_loop(0, HALF // 2 - 1, body, 0)
        pltpu.async_copy(xs_hbm.at[rowi_v.at[HALF - 1]], rows1_v, sem1)
        pltpu.make_async_copy(xs_hbm.at[rowi_v.at[HALF - 2]], rows0_v, sem0).wait()
        pltpu.sync_copy(rows0_v, acc.at[coli_v.at[HALF - 2]], add=True)
        pltpu.make_async_copy(xs_hbm.at[rowi_v.at[HALF - 1]], rows1_v, sem1).wait()
        pltpu.sync_copy(rows1_v, acc.at[coli_v.at[HALF - 1]], add=True)
        return carry

    lax.fori_loop(0, 2, half_body, 0)
    plsc.subcore_barrier()
    pltpu.sync_copy(acc.at[pl.ds(sid * RT, RT)],
                    g_hbm.at[cid, pl.ds(sid * RT, RT)])


_BLK = 2000


def _scale_body(hist_ref, x_ref, xs_ref, dinv_ref):
    deg = hist_ref[0, :, 0:1] + hist_ref[1, :, 0:1] + 1.0
    dinv = lax.rsqrt(deg)
    xs_ref[...] = dinv * x_ref[...]
    dinv_ref[...] = jnp.broadcast_to(dinv, (_BLK, 16))


def _mid_body(g_ref, xs_ref, dinv_ref, w_ref, b_ref, xs2_ref):
    dinv = dinv_ref[:, 0:1]
    p = dinv * (g_ref[0] + g_ref[1] + xs_ref[...])
    h = jnp.dot(p, w_ref[...], preferred_element_type=jnp.float32) + b_ref[...]
    xs2_ref[...] = dinv * h


def _out_body(g_ref, xs2_ref, dinv_ref, w2_ref, b2_ref, w3_ref, b3_ref,
              m_ref, s_ref):
    dinv = dinv_ref[:, 0:1]
    p = dinv * (g_ref[0] + g_ref[1] + xs2_ref[...])
    m_ref[...] = jnp.maximum(
        jnp.dot(p, w2_ref[...], preferred_element_type=jnp.float32) + b2_ref[...], 0.0)
    s_ref[...] = jnp.maximum(
        jnp.dot(p, w3_ref[...], preferred_element_type=jnp.float32) + b3_ref[...], 0.0)


def _row_spec(i):
    return (i, 0)


def _pair_spec(i):
    return (0, i, 0)


def _full_spec(i):
    return (0, 0)


def kernel(x, edge_index, W1, b1, W2, b2, W3, b3):
    ei = edge_index.astype(jnp.int32)
    pad = EPAD - E
    padi = jnp.arange(pad, dtype=jnp.int32) % NJ
    row3 = jnp.concatenate([ei[0], padi]).reshape(NW, CPW, CH)
    col3 = jnp.concatenate([ei[1], N + padi]).reshape(NW, CPW, CH)
    ones16 = jnp.concatenate(
        [jnp.ones((CH, 1), jnp.float32), jnp.zeros((CH, 15), jnp.float32)], axis=1)
    z16 = jnp.zeros((NP, 16), jnp.float32)
    z128 = jnp.zeros((NP, F), jnp.float32)
    b1r, b2r, b3r = b1.reshape(1, F), b2.reshape(1, F), b3.reshape(1, F)

    hist = _degree_kernel(col3, ones16, z16)

    grid = (N // _BLK,)
    xs1, dinv16 = pl.pallas_call(
        _scale_body,
        grid=grid,
        in_specs=[pl.BlockSpec((NC, _BLK, 16), _pair_spec),
                  pl.BlockSpec((_BLK, F), _row_spec)],
        out_specs=[pl.BlockSpec((_BLK, F), _row_spec),
                   pl.BlockSpec((_BLK, 16), _row_spec)],
        out_shape=[jax.ShapeDtypeStruct((N, F), jnp.float32),
                   jax.ShapeDtypeStruct((N, 16), jnp.float32)],
    )(hist, x)

    g1 = _prop_kernel(xs1, row3, col3, z128)

    xs2 = pl.pallas_call(
        _mid_body,
        grid=grid,
        in_specs=[pl.BlockSpec((NC, _BLK, F), _pair_spec),
                  pl.BlockSpec((_BLK, F), _row_spec),
                  pl.BlockSpec((_BLK, 16), _row_spec),
                  pl.BlockSpec((F, F), _full_spec),
                  pl.BlockSpec((1, F), _full_spec)],
        out_specs=pl.BlockSpec((_BLK, F), _row_spec),
        out_shape=jax.ShapeDtypeStruct((N, F), jnp.float32),
    )(g1, xs1, dinv16, W1, b1r)

    g2 = _prop_kernel(xs2, row3, col3, z128)

    m, s = pl.pallas_call(
        _out_body,
        grid=grid,
        in_specs=[pl.BlockSpec((NC, _BLK, F), _pair_spec),
                  pl.BlockSpec((_BLK, F), _row_spec),
                  pl.BlockSpec((_BLK, 16), _row_spec),
                  pl.BlockSpec((F, F), _full_spec),
                  pl.BlockSpec((1, F), _full_spec),
                  pl.BlockSpec((F, F), _full_spec),
                  pl.BlockSpec((1, F), _full_spec)],
        out_specs=[pl.BlockSpec((_BLK, F), _row_spec),
                   pl.BlockSpec((_BLK, F), _row_spec)],
        out_shape=[jax.ShapeDtypeStruct((N, F), jnp.float32),
                   jax.ShapeDtypeStruct((N, F), jnp.float32)],
    )(g2, xs2, dinv16, W2, b2r, W3, b3r)

    return (m, s)


# speedup vs baseline: 3.7612x; 1.0105x over previous
import functools

import jax
import jax.numpy as jnp
from jax import lax
from jax.experimental import pallas as pl
from jax.experimental.pallas import tpu as pltpu
from jax.experimental.pallas import tpu_sc as plsc

N = 10000
F = 128
E = 320000
NC, NS = 2, 16
NW = NC * NS
CH = 128
CPW = 80
EPAD = NW * CPW * CH
HALF = CPW // 2
NP = 10240
NJ = NP - N
RT = NP // NS

_sc_mesh = plsc.VectorSubcoreMesh(
    core_axis_name="c", subcore_axis_name="s", num_cores=NC, num_subcores=NS)


@functools.partial(
    pl.kernel,
    out_type=jax.ShapeDtypeStruct((NC, NP, 16), jnp.float32),
    mesh=_sc_mesh,
    scratch_types=[
        pltpu.VMEM((CPW, CH), jnp.int32),
        pltpu.VMEM((CH, 16), jnp.float32),
        pltpu.SemaphoreType.DMA,
        pltpu.VMEM_SHARED((NP, 16), jnp.float32),
    ],
)
def _degree_kernel(col_hbm, ones_hbm, z16_hbm, hist_hbm, idx_v, ones_v, sem,
                   acc):
    cid = lax.axis_index("c")
    sid = lax.axis_index("s")
    wid = sid * NC + cid
    pltpu.sync_copy(z16_hbm.at[pl.ds(sid * RT, RT)], acc.at[pl.ds(sid * RT, RT)])
    pltpu.sync_copy(ones_hbm, ones_v)
    pltpu.sync_copy(col_hbm.at[wid], idx_v)
    plsc.subcore_barrier()

    def body(grp, c):
        for k in range(16):
            pltpu.async_copy(ones_v, acc.at[idx_v.at[grp * 16 + k]], sem,
                             add=True)
        for k in range(16):
            pltpu.make_async_copy(ones_v, acc.at[idx_v.at[grp * 16 + k]],
                                  sem).wait()
        return c

    lax.fori_loop(0, CPW // 16, body, 0)
    plsc.subcore_barrier()
    pltpu.sync_copy(acc.at[pl.ds(sid * RT, RT)],
                    hist_hbm.at[cid, pl.ds(sid * RT, RT)])


@functools.partial(
    pl.kernel,
    out_type=jax.ShapeDtypeStruct((NC, NP, F), jnp.float32),
    mesh=_sc_mesh,
    scratch_types=[
        pltpu.VMEM((HALF, CH), jnp.int32),
        pltpu.VMEM((HALF, CH), jnp.int32),
        pltpu.VMEM((CH, F), jnp.float32),
        pltpu.VMEM((CH, F), jnp.float32),
        pltpu.SemaphoreType.DMA,
        pltpu.SemaphoreType.DMA,
        pltpu.VMEM_SHARED((NP, F), jnp.float32),
    ],
)
def _prop_kernel(xs_hbm, row_hbm, col_hbm, z128_hbm, g_hbm,
                 rowi_v, coli_v, rows0_v, rows1_v, sem0, sem1, acc):
    cid = lax.axis_index("c")
    sid = lax.axis_index("s")
    wid = sid * NC + cid
    pltpu.sync_copy(z128_hbm.at[pl.ds(sid * RT, RT)], acc.at[pl.ds(sid * RT, RT)])
    plsc.subcore_barrier()

    def half_body(h, carry):
        pltpu.sync_copy(row_hbm.at[wid, pl.ds(h * HALF, HALF)], rowi_v)
        pltpu.sync_copy(col_hbm.at[wid, pl.ds(h * HALF, HALF)], coli_v)
        pltpu.async_copy(xs_hbm.at[rowi_v.at[0]], rows0_v, sem0)

        def body(i, c):
            j = 2 * i
            pltpu.async_copy(xs_hbm.at[rowi_v.at[j + 1]], rows1_v, sem1)
            pltpu.make_async_copy(xs_hbm.at[rowi_v.at[j]], rows0_v, sem0).wait()
            pltpu.sync_copy(rows0_v, acc.at[coli_v.at[j]], add=True)
            pltpu.async_copy(xs_hbm.at[rowi_v.at[j + 2]], rows0_v, sem0)
            pltpu.make_async_copy(xs_hbm.at[rowi_v.at[j + 1]], rows1_v, sem1).wait()
            pltpu.sync_copy(rows1_v, acc.at[coli_v.at[j + 1]], add=True)
            return c

        lax.fori_loop(0, HALF // 2 - 1, body, 0)
        pltpu.async_copy(xs_hbm.at[rowi_v.at[HALF - 1]], rows1_v, sem1)
        pltpu.make_async_copy(xs_hbm.at[rowi_v.at[HALF - 2]], rows0_v, sem0).wait()
        pltpu.sync_copy(rows0_v, acc.at[coli_v.at[HALF - 2]], add=True)
        pltpu.make_async_copy(xs_hbm.at[rowi_v.at[HALF - 1]], rows1_v, sem1).wait()
        pltpu.sync_copy(rows1_v, acc.at[coli_v.at[HALF - 1]], add=True)
        return carry

    lax.fori_loop(0, 2, half_body, 0)
    plsc.subcore_barrier()
    pltpu.sync_copy(acc.at[pl.ds(sid * RT, RT)],
                    g_hbm.at[cid, pl.ds(sid * RT, RT)])


_BLK = 2000


def _scale_body(hist_ref, x_ref, xs_ref, dinv_ref):
    deg = hist_ref[0, :, 0:1] + hist_ref[1, :, 0:1] + 1.0
    dinv = lax.rsqrt(deg)
    xs_ref[...] = dinv * x_ref[...]
    dinv_ref[...] = jnp.broadcast_to(dinv, (_BLK, 16))


def _mid_body(g_ref, xs_ref, dinv_ref, w_ref, b_ref, xs2_ref):
    dinv = dinv_ref[:, 0:1]
    p = dinv * (g_ref[0] + g_ref[1] + xs_ref[...])
    h = jnp.dot(p, w_ref[...], preferred_element_type=jnp.float32) + b_ref[...]
    xs2_ref[...] = dinv * h


def _out_body(g_ref, xs2_ref, dinv_ref, w2_ref, b2_ref, w3_ref, b3_ref,
              m_ref, s_ref):
    dinv = dinv_ref[:, 0:1]
    p = dinv * (g_ref[0] + g_ref[1] + xs2_ref[...])
    m_ref[...] = jnp.maximum(
        jnp.dot(p, w2_ref[...], preferred_element_type=jnp.float32) + b2_ref[...], 0.0)
    s_ref[...] = jnp.maximum(
        jnp.dot(p, w3_ref[...], preferred_element_type=jnp.float32) + b3_ref[...], 0.0)


def _row_spec(i):
    return (i, 0)


def _pair_spec(i):
    return (0, i, 0)


def _full_spec(i):
    return (0, 0)


def kernel(x, edge_index, W1, b1, W2, b2, W3, b3):
    ei = edge_index.astype(jnp.int32)
    pad = EPAD - E
    padi = jnp.arange(pad, dtype=jnp.int32) % NJ
    row3 = jnp.concatenate([ei[0], padi]).reshape(NW, CPW, CH)
    col3 = jnp.concatenate([ei[1], N + padi]).reshape(NW, CPW, CH)
    ones16 = jnp.concatenate(
        [jnp.ones((CH, 1), jnp.float32), jnp.zeros((CH, 15), jnp.float32)], axis=1)
    z16 = jnp.zeros((NP, 16), jnp.float32)
    z128 = jnp.zeros((NP, F), jnp.float32)
    b1r, b2r, b3r = b1.reshape(1, F), b2.reshape(1, F), b3.reshape(1, F)

    hist = _degree_kernel(col3, ones16, z16)

    grid = (N // _BLK,)
    xs1, dinv16 = pl.pallas_call(
        _scale_body,
        grid=grid,
        in_specs=[pl.BlockSpec((NC, _BLK, 16), _pair_spec),
                  pl.BlockSpec((_BLK, F), _row_spec)],
        out_specs=[pl.BlockSpec((_BLK, F), _row_spec),
                   pl.BlockSpec((_BLK, 16), _row_spec)],
        out_shape=[jax.ShapeDtypeStruct((N, F), jnp.float32),
                   jax.ShapeDtypeStruct((N, 16), jnp.float32)],
    )(hist, x)

    g1 = _prop_kernel(xs1, row3, col3, z128)

    xs2 = pl.pallas_call(
        _mid_body,
        grid=grid,
        in_specs=[pl.BlockSpec((NC, _BLK, F), _pair_spec),
                  pl.BlockSpec((_BLK, F), _row_spec),
                  pl.BlockSpec((_BLK, 16), _row_spec),
                  pl.BlockSpec((F, F), _full_spec),
                  pl.BlockSpec((1, F), _full_spec)],
        out_specs=pl.BlockSpec((_BLK, F), _row_spec),
        out_shape=jax.ShapeDtypeStruct((N, F), jnp.float32),
    )(g1, xs1, dinv16, W1, b1r)

    g2 = _prop_kernel(xs2, row3, col3, z128)

    m, s = pl.pallas_call(
        _out_body,
        grid=grid,
        in_specs=[pl.BlockSpec((NC, _BLK, F), _pair_spec),
                  pl.BlockSpec((_BLK, F), _row_spec),
                  pl.BlockSpec((_BLK, 16), _row_spec),
                  pl.BlockSpec((F, F), _full_spec),
                  pl.BlockSpec((1, F), _full_spec),
                  pl.BlockSpec((F, F), _full_spec),
                  pl.BlockSpec((1, F), _full_spec)],
        out_specs=[pl.BlockSpec((_BLK, F), _row_spec),
                   pl.BlockSpec((_BLK, F), _row_spec)],
        out_shape=[jax.ShapeDtypeStruct((N, F), jnp.float32),
                   jax.ShapeDtypeStruct((N, F), jnp.float32)],
    )(g2, xs2, dinv16, W2, b2r, W3, b3r)

    return (m, s)
